# baseline (speedup 1.0000x reference)
import functools

import jax
import jax.numpy as jnp
from jax import lax
from jax.experimental import pallas as pl
from jax.experimental.pallas import tpu as pltpu
from jax.experimental.pallas import tpu_sc as plsc

B = 16
L = 2048
D = 128
NEXT_TOKEN_ID = 5
EPS = 1e-05

N = B * L
NC = 2
NS = 16
NW = NC * NS
PER_W = N // NW
CHUNK = 128
N_CHUNKS = PER_W // CHUNK


def _shift_right(x, s, fill):
    pad = jnp.full((x.shape[0], s), fill, dtype=x.dtype)
    return jnp.concatenate([pad, x[:, : x.shape[1] - s]], axis=1)


def _indices_kernel(state_ref, instr_ref, arg_ref):
    state = state_ref[...]
    eq = (state == NEXT_TOKEN_ID).astype(jnp.int32)
    csum = eq
    s = 1
    while s < L:
        csum = csum + _shift_right(csum, s, 0)
        s *= 2
    instr_ref[...] = csum - eq
    pos = lax.broadcasted_iota(jnp.int32, (B, L), 1)
    m = jnp.where(eq > 0, pos + 1, 0)
    s = 1
    while s < L:
        m = jnp.maximum(m, _shift_right(m, s, 0))
        s *= 2
    seg_start = _shift_right(m, 1, 0)
    arg_ref[...] = pos - seg_start


def _compute_indices(state):
    return pl.pallas_call(
        _indices_kernel,
        out_shape=(
            jax.ShapeDtypeStruct((B, L), jnp.int32),
            jax.ShapeDtypeStruct((B, L), jnp.int32),
        ),
    )(state)


NSLOT = 2


def _gather_sum_kernel(state_hbm, instr_hbm, arg_hbm,
                       tok_tab, ins_tab, arg_tab, out_hbm,
                       sidx, iidx, aidx, tbufs, ibufs, abufs, gsems, osems):
    wid = lax.axis_index("s") * NC + lax.axis_index("c")
    base = pl.multiple_of(wid * PER_W, 256)
    pltpu.sync_copy(state_hbm.at[pl.ds(base, PER_W)], sidx)
    pltpu.sync_copy(instr_hbm.at[pl.ds(base, PER_W)], iidx)
    pltpu.sync_copy(arg_hbm.at[pl.ds(base, PER_W)], aidx)

    scat = {}

    def fire(c):
        s = c % NSLOT

        def grp(g, carry):
            off = c * CHUNK + 16 * g
            tv = sidx[pl.ds(off, 16)]
            iv = iidx[pl.ds(off, 16)]
            av = aidx[pl.ds(off, 16)]
            for k in range(16):
                r = 16 * g + k
                pltpu.async_copy(
                    tok_tab.at[pl.ds(tv[k], 1)], tbufs[s].at[pl.ds(r, 1)],
                    gsems[s])
                pltpu.async_copy(
                    ins_tab.at[pl.ds(iv[k], 1)], ibufs[s].at[pl.ds(r, 1)],
                    gsems[s])
                pltpu.async_copy(
                    arg_tab.at[pl.ds(av[k], 1)], abufs[s].at[pl.ds(r, 1)],
                    gsems[s])
            return carry

        lax.fori_loop(0, CHUNK // 16, grp, jnp.int32(0))
        return None

    def drain(c):
        s = c % NSLOT
        dummy = tok_tab.at[pl.ds(0, CHUNK)]
        pltpu.make_async_copy(dummy, tbufs[s], gsems[s]).wait()
        pltpu.make_async_copy(dummy, ibufs[s], gsems[s]).wait()
        pltpu.make_async_copy(dummy, abufs[s], gsems[s]).wait()

    pend = {c: fire(c) for c in range(min(NSLOT, N_CHUNKS))}
    for c in range(N_CHUNKS):
        s = c % NSLOT
        pend.pop(c)
        drain(c)

        def row(r, carry):
            for k in range(D // 16):
                tbufs[s][r, pl.ds(16 * k, 16)] = (
                    tbufs[s][r, pl.ds(16 * k, 16)]
                    + ibufs[s][r, pl.ds(16 * k, 16)]
                    + abufs[s][r, pl.ds(16 * k, 16)])
            return carry

        lax.fori_loop(0, CHUNK, row, jnp.int32(0))
        scat[c] = pltpu.async_copy(
            tbufs[s], out_hbm.at[pl.ds(base + c * CHUNK, CHUNK)], osems[s])
        if c + NSLOT < N_CHUNKS:
            scat.pop(c).wait()
            pend[c + NSLOT] = fire(c + NSLOT)
    for c in sorted(scat):
        scat[c].wait()


_gather_sum = functools.partial(
    pl.kernel,
    out_type=jax.ShapeDtypeStruct((N, D), jnp.float32),
    mesh=plsc.VectorSubcoreMesh(core_axis_name="c", subcore_axis_name="s"),
    scratch_types=[
        pltpu.VMEM((PER_W,), jnp.int32),
        pltpu.VMEM((PER_W,), jnp.int32),
        pltpu.VMEM((PER_W,), jnp.int32),
        [pltpu.VMEM((CHUNK, D), jnp.float32) for _ in range(NSLOT)],
        [pltpu.VMEM((CHUNK, D), jnp.float32) for _ in range(NSLOT)],
        [pltpu.VMEM((CHUNK, D), jnp.float32) for _ in range(NSLOT)],
        [pltpu.SemaphoreType.DMA for _ in range(NSLOT)],
        [pltpu.SemaphoreType.DMA for _ in range(NSLOT)],
    ],
)(_gather_sum_kernel)


LN_BLOCK = 1024


def _ln_kernel(x_ref, w_ref, b_ref, o_ref):
    x = x_ref[...]
    mean = jnp.mean(x, axis=-1, keepdims=True)
    d = x - mean
    var = jnp.mean(d * d, axis=-1, keepdims=True)
    rstd = lax.rsqrt(var + EPS)
    o_ref[...] = d * rstd * w_ref[...] + b_ref[...]


def _layernorm(x, w, b):
    return pl.pallas_call(
        _ln_kernel,
        grid=(N // LN_BLOCK,),
        in_specs=[
            pl.BlockSpec((LN_BLOCK, D), lambda i: (i, 0)),
            pl.BlockSpec((1, D), lambda i: (0, 0)),
            pl.BlockSpec((1, D), lambda i: (0, 0)),
        ],
        out_specs=pl.BlockSpec((LN_BLOCK, D), lambda i: (i, 0)),
        out_shape=jax.ShapeDtypeStruct((N, D), jnp.float32),
    )(x, w.reshape(1, D), b.reshape(1, D))


def kernel(state, token_table, instr_table, arg_table, ln_weight, ln_bias):
    instructions, arguments = _compute_indices(state)
    summed = _gather_sum(
        state.reshape(N), instructions.reshape(N), arguments.reshape(N),
        token_table, instr_table, arg_table)
    out = _layernorm(summed, ln_weight, ln_bias)
    return out.reshape(B, L, D)


# speedup vs baseline: 10.1856x; 10.1856x over previous
import functools

import jax
import jax.numpy as jnp
from jax import lax
from jax.experimental import pallas as pl
from jax.experimental.pallas import tpu as pltpu
from jax.experimental.pallas import tpu_sc as plsc

B = 16
L = 2048
D = 128
NEXT_TOKEN_ID = 5
EPS = 1e-05

N = B * L
NC = 2
NS = 16
NW = NC * NS
PER_W = N // NW
CHUNK = 128
N_CHUNKS = PER_W // CHUNK


def _shift_right(x, s, fill):
    pad = jnp.full((x.shape[0], s), fill, dtype=x.dtype)
    return jnp.concatenate([pad, x[:, : x.shape[1] - s]], axis=1)


def _indices_kernel(state_ref, instr_ref, arg_ref):
    state = state_ref[...]
    eq = (state == NEXT_TOKEN_ID).astype(jnp.int32)
    csum = eq
    s = 1
    while s < L:
        csum = csum + _shift_right(csum, s, 0)
        s *= 2
    instr_ref[...] = csum - eq
    pos = lax.broadcasted_iota(jnp.int32, (B, L), 1)
    m = jnp.where(eq > 0, pos + 1, 0)
    s = 1
    while s < L:
        m = jnp.maximum(m, _shift_right(m, s, 0))
        s *= 2
    seg_start = _shift_right(m, 1, 0)
    arg_ref[...] = pos - seg_start


def _compute_indices(state):
    return pl.pallas_call(
        _indices_kernel,
        out_shape=(
            jax.ShapeDtypeStruct((B, L), jnp.int32),
            jax.ShapeDtypeStruct((B, L), jnp.int32),
        ),
    )(state)


NSLOT = 2
NV = D // 16

_GDN = lax.GatherDimensionNumbers(
    offset_dims=(), collapsed_slice_dims=(0,), start_index_map=(0,))


def _hsum(v, lane16):
    for sh in (8, 4, 2, 1):
        perm = lax.rem(lane16 + jnp.int32(sh), jnp.int32(16))
        v = v + lax.gather(v, perm[:, None], _GDN, (1,),
                           mode=lax.GatherScatterMode.PROMISE_IN_BOUNDS)
    return v


def _rsqrt_ladder(x):
    c = jnp.full(x.shape, 1.0, jnp.float32)
    for p in (8, 4, 2, 1):
        big = x >= jnp.float32(4.0 ** p)
        x = jnp.where(big, x * jnp.float32(4.0 ** -p), x)
        c = jnp.where(big, c * jnp.float32(2.0 ** -p), c)
    for p in (8, 4, 2, 1):
        small = x < jnp.float32(4.0 ** (1 - p))
        x = jnp.where(small, x * jnp.float32(4.0 ** p), x)
        c = jnp.where(small, c * jnp.float32(2.0 ** p), c)
    y = jnp.float32(1.27889018) + x * (
        jnp.float32(-0.3700147) + x * jnp.float32(0.04463214))
    for _ in range(3):
        y = y * (jnp.float32(1.5) - jnp.float32(0.5) * x * y * y)
    return y * c


def _ln_store(dst, r, xs, wr, br, lane16):
    tot = xs[0]
    sq = xs[0] * xs[0]
    for k in range(1, NV):
        tot = tot + xs[k]
        sq = sq + xs[k] * xs[k]
    inv_d = jnp.float32(1.0 / D)
    mean = _hsum(tot, lane16) * inv_d
    ex2 = _hsum(sq, lane16) * inv_d
    var = ex2 - mean * mean
    rstd = _rsqrt_ladder(var + jnp.float32(EPS))
    for k in range(NV):
        dst[r, pl.ds(16 * k, 16)] = (xs[k] - mean) * rstd * wr[k] + br[k]


def _gather_sum_kernel(state_hbm, instr_hbm, arg_hbm, meta_hbm,
                       tok_tab, ins_tab, arg_tab, w_hbm, b_hbm, out_hbm,
                       sidx, iidx, aidx, metav, wv, bv, tbufs, ibufs, abufs,
                       inrs, tsems, asems, osems):
    wid = lax.axis_index("s") * NC + lax.axis_index("c")
    base = pl.multiple_of(wid * PER_W, 256)
    pltpu.sync_copy(state_hbm.at[pl.ds(base, PER_W)], sidx)
    pltpu.sync_copy(instr_hbm.at[pl.ds(base, PER_W)], iidx)
    pltpu.sync_copy(arg_hbm.at[pl.ds(base, PER_W)], aidx)
    pltpu.sync_copy(meta_hbm.at[pl.ds(wid * (N_CHUNKS * 16), N_CHUNKS * 16)],
                    metav)
    pltpu.sync_copy(w_hbm, wv)
    pltpu.sync_copy(b_hbm, bv)
    lane16 = lax.broadcasted_iota(jnp.int32, (16,), 0)
    wr = [wv[pl.ds(16 * k, 16)] for k in range(NV)]
    br = [bv[pl.ds(16 * k, 16)] for k in range(NV)]

    scat = {}

    def fire(c):
        s = c % NSLOT

        def grp(g, carry):
            tv = sidx[pl.ds(c * CHUNK + 16 * g, 16)]
            for k in range(16):
                pltpu.async_copy(
                    tok_tab.at[pl.ds(tv[k], 1)],
                    tbufs[s].at[pl.ds(16 * g + k, 1)], tsems[s])
            return carry

        lax.fori_loop(0, CHUNK // 16, grp, jnp.int32(0))

        mvc = metav[pl.ds(16 * c, 16)]
        bnd = mvc[2]

        @pl.when(bnd == 0)
        def _fast():
            pltpu.async_copy(ins_tab.at[pl.ds(mvc[0], 1)], inrs[s], asems[s])
            pltpu.async_copy(
                arg_tab.at[pl.ds(pl.multiple_of(mvc[1], 8), CHUNK + 8)],
                abufs[s], asems[s])

        @pl.when(bnd != 0)
        def _slow():
            def grp2(g, carry):
                iv = iidx[pl.ds(c * CHUNK + 16 * g, 16)]
                av = aidx[pl.ds(c * CHUNK + 16 * g, 16)]
                for k in range(16):
                    r = 16 * g + k
                    pltpu.async_copy(
                        ins_tab.at[pl.ds(iv[k], 1)],
                        ibufs[s].at[pl.ds(r, 1)], asems[s])
                    pltpu.async_copy(
                        arg_tab.at[pl.ds(av[k], 1)],
                        abufs[s].at[pl.ds(r, 1)], asems[s])
                return carry

            lax.fori_loop(0, CHUNK // 16, grp2, jnp.int32(0))

    for c in range(min(NSLOT, N_CHUNKS)):
        fire(c)
    for c in range(N_CHUNKS):
        s = c % NSLOT
        pltpu.make_async_copy(
            tok_tab.at[pl.ds(0, CHUNK)], tbufs[s], tsems[s]).wait()
        mvc = metav[pl.ds(16 * c, 16)]
        bnd = mvc[2]

        @pl.when(bnd == 0)
        def _drain_fast():
            pltpu.make_async_copy(
                ins_tab.at[pl.ds(0, 1)], inrs[s], asems[s]).wait()
            pltpu.make_async_copy(
                arg_tab.at[pl.ds(0, CHUNK + 8)], abufs[s], asems[s]).wait()

            iv8 = [inrs[s][0, pl.ds(16 * k, 16)] for k in range(D // 16)]
            off = mvc[3]

            def rowf(r, carry):
                xs = [tbufs[s][r, pl.ds(16 * k, 16)]
                      + abufs[s][r + off, pl.ds(16 * k, 16)] + iv8[k]
                      for k in range(NV)]
                _ln_store(tbufs[s], r, xs, wr, br, lane16)
                return carry

            lax.fori_loop(0, CHUNK, rowf, jnp.int32(0))

        @pl.when(bnd != 0)
        def _drain_slow():
            pltpu.make_async_copy(
                ins_tab.at[pl.ds(0, CHUNK)], ibufs[s], asems[s]).wait()
            pltpu.make_async_copy(
                arg_tab.at[pl.ds(0, CHUNK)],
                abufs[s].at[pl.ds(0, CHUNK)], asems[s]).wait()

            def rows(r, carry):
                xs = [tbufs[s][r, pl.ds(16 * k, 16)]
                      + abufs[s][r, pl.ds(16 * k, 16)]
                      + ibufs[s][r, pl.ds(16 * k, 16)]
                      for k in range(NV)]
                _ln_store(tbufs[s], r, xs, wr, br, lane16)
                return carry

            lax.fori_loop(0, CHUNK, rows, jnp.int32(0))

        scat[c] = pltpu.async_copy(
            tbufs[s], out_hbm.at[pl.ds(base + c * CHUNK, CHUNK)], osems[s])
        if c + NSLOT < N_CHUNKS:
            scat.pop(c).wait()
            fire(c + NSLOT)
    for c in sorted(scat):
        scat[c].wait()


_gather_sum = functools.partial(
    pl.kernel,
    out_type=jax.ShapeDtypeStruct((N, D), jnp.float32),
    mesh=plsc.VectorSubcoreMesh(core_axis_name="c", subcore_axis_name="s"),
    scratch_types=[
        pltpu.VMEM((PER_W,), jnp.int32),
        pltpu.VMEM((PER_W,), jnp.int32),
        pltpu.VMEM((PER_W,), jnp.int32),
        pltpu.VMEM((N_CHUNKS * 16,), jnp.int32),
        pltpu.VMEM((D,), jnp.float32),
        pltpu.VMEM((D,), jnp.float32),
        [pltpu.VMEM((CHUNK, D), jnp.float32) for _ in range(NSLOT)],
        [pltpu.VMEM((CHUNK, D), jnp.float32) for _ in range(NSLOT)],
        [pltpu.VMEM((CHUNK + 8, D), jnp.float32) for _ in range(NSLOT)],
        [pltpu.VMEM((1, D), jnp.float32) for _ in range(NSLOT)],
        [pltpu.SemaphoreType.DMA for _ in range(NSLOT)],
        [pltpu.SemaphoreType.DMA for _ in range(NSLOT)],
        [pltpu.SemaphoreType.DMA for _ in range(NSLOT)],
    ],
)(_gather_sum_kernel)


LN_BLOCK = 1024


def _ln_kernel(x_ref, w_ref, b_ref, o_ref):
    x = x_ref[...]
    mean = jnp.mean(x, axis=-1, keepdims=True)
    d = x - mean
    var = jnp.mean(d * d, axis=-1, keepdims=True)
    rstd = lax.rsqrt(var + EPS)
    o_ref[...] = d * rstd * w_ref[...] + b_ref[...]


def _layernorm(x, w, b):
    return pl.pallas_call(
        _ln_kernel,
        grid=(N // LN_BLOCK,),
        in_specs=[
            pl.BlockSpec((LN_BLOCK, D), lambda i: (i, 0)),
            pl.BlockSpec((1, D), lambda i: (0, 0)),
            pl.BlockSpec((1, D), lambda i: (0, 0)),
        ],
        out_specs=pl.BlockSpec((LN_BLOCK, D), lambda i: (i, 0)),
        out_shape=jax.ShapeDtypeStruct((N, D), jnp.float32),
    )(x, w.reshape(1, D), b.reshape(1, D))


def kernel(state, token_table, instr_table, arg_table, ln_weight, ln_bias):
    instructions, arguments = _compute_indices(state)
    i0 = instructions[:, ::CHUNK].reshape(-1)
    a0 = arguments[:, ::CHUNK].reshape(-1)
    bnd = (instructions[:, CHUNK - 1::CHUNK] != instructions[:, ::CHUNK])
    meta = jnp.zeros((N // CHUNK, 16), jnp.int32)
    a0_al = (a0 // 8) * 8
    meta = meta.at[:, 0].set(i0).at[:, 1].set(a0_al)
    meta = meta.at[:, 2].set(bnd.reshape(-1).astype(jnp.int32))
    meta = meta.at[:, 3].set(a0 - a0_al)
    arg_padded = jnp.concatenate(
        [arg_table, jnp.zeros((8, D), jnp.float32)], axis=0)
    out = _gather_sum(
        state.reshape(N), instructions.reshape(N), arguments.reshape(N),
        meta.reshape(-1), token_table, instr_table, arg_padded,
        ln_weight, ln_bias)
    return out.reshape(B, L, D)


# speedup vs baseline: 11.3539x; 1.1147x over previous
import functools

import jax
import jax.numpy as jnp
from jax import lax
from jax.experimental import pallas as pl
from jax.experimental.pallas import tpu as pltpu
from jax.experimental.pallas import tpu_sc as plsc

B = 16
L = 2048
D = 128
NEXT_TOKEN_ID = 5
EPS = 1e-05

N = B * L
NC = 2
NS = 16
NW = NC * NS
PER_W = N // NW
HALF = L // 2
CHUNK = 128
N_CHUNKS = PER_W // CHUNK
NSLOT = 2
NV = D // 16

_GDN = lax.GatherDimensionNumbers(
    offset_dims=(), collapsed_slice_dims=(0,), start_index_map=(0,))


def _rot(v, lane16, sh):
    perm = lax.rem(lane16 + jnp.int32(sh), jnp.int32(16))
    return lax.gather(v, perm[:, None], _GDN, (1,),
                      mode=lax.GatherScatterMode.PROMISE_IN_BOUNDS)


def _hsum(v, lane16):
    for sh in (8, 4, 2, 1):
        v = v + _rot(v, lane16, sh)
    return v


def _hmax(v, lane16):
    for sh in (8, 4, 2, 1):
        v = jnp.maximum(v, _rot(v, lane16, sh))
    return v


def _cumsum16(x, lane16):
    for sh in (1, 2, 4, 8):
        sl = _rot(x, lane16, 16 - sh)
        x = x + jnp.where(lane16 >= jnp.int32(sh), sl, jnp.int32(0))
    return x


def _cummax16(x, lane16):
    for sh in (1, 2, 4, 8):
        sl = _rot(x, lane16, 16 - sh)
        x = jnp.maximum(x, jnp.where(lane16 >= jnp.int32(sh), sl,
                                     jnp.int32(0)))
    return x


def _rsqrt_ladder(x):
    c = jnp.full(x.shape, 1.0, jnp.float32)
    for p in (8, 4, 2, 1):
        big = x >= jnp.float32(4.0 ** p)
        x = jnp.where(big, x * jnp.float32(4.0 ** -p), x)
        c = jnp.where(big, c * jnp.float32(2.0 ** -p), c)
    for p in (8, 4, 2, 1):
        small = x < jnp.float32(4.0 ** (1 - p))
        x = jnp.where(small, x * jnp.float32(4.0 ** p), x)
        c = jnp.where(small, c * jnp.float32(2.0 ** p), c)
    y = jnp.float32(1.27889018) + x * (
        jnp.float32(-0.3700147) + x * jnp.float32(0.04463214))
    for _ in range(3):
        y = y * (jnp.float32(1.5) - jnp.float32(0.5) * x * y * y)
    return y * c


def _ln_store(dst, r, xs, wr, br, lane16):
    tot = xs[0]
    sq = xs[0] * xs[0]
    for k in range(1, NV):
        tot = tot + xs[k]
        sq = sq + xs[k] * xs[k]
    inv_d = jnp.float32(1.0 / D)
    mean = _hsum(tot, lane16) * inv_d
    ex2 = _hsum(sq, lane16) * inv_d
    var = ex2 - mean * mean
    rstd = _rsqrt_ladder(var + jnp.float32(EPS))
    for k in range(NV):
        dst[r, pl.ds(16 * k, 16)] = (xs[k] - mean) * rstd * wr[k] + br[k]


def _fused_kernel(state_hbm, tok_tab, ins_tab, arg_tab, w_hbm, b_hbm,
                  out_hbm, sv, pre, iidx, aidx, wv, bv, tbufs, ibufs, abufs,
                  inrs, tsems, asems, osems):
    wid = lax.axis_index("s") * NC + lax.axis_index("c")
    base = pl.multiple_of(wid * PER_W, 256)
    half = lax.rem(wid, jnp.int32(2))
    rowbase = half * HALF
    lane16 = lax.broadcasted_iota(jnp.int32, (16,), 0)

    pltpu.sync_copy(state_hbm.at[pl.ds(base, PER_W)], sv)
    pre_base = pl.multiple_of(jnp.maximum(base - HALF, 0), 256)
    pltpu.sync_copy(state_hbm.at[pl.ds(pre_base, HALF)], pre)
    pltpu.sync_copy(w_hbm, wv)
    pltpu.sync_copy(b_hbm, bv)
    wr = [wv[pl.ds(16 * k, 16)] for k in range(NV)]
    br = [bv[pl.ds(16 * k, 16)] for k in range(NV)]

    def pre_step(j, carry):
        cnt_v, seg_v = carry
        rp = jnp.int32(16) * j + lane16
        e = (jnp.where(pre[pl.ds(16 * j, 16)] == NEXT_TOKEN_ID,
                       jnp.int32(1), jnp.int32(0))
             * jnp.where(rp <= jnp.int32(HALF - 2), jnp.int32(1),
                         jnp.int32(0)))
        cnt_v = cnt_v + e
        seg_v = jnp.maximum(seg_v, jnp.where(e > 0, rp + jnp.int32(1),
                                             jnp.int32(0)))
        return cnt_v, seg_v

    z = jnp.zeros((16,), jnp.int32)
    cnt_v, seg_v = lax.fori_loop(0, HALF // 16, pre_step, (z, z))
    cnt = half * _hsum(cnt_v, lane16)[0]
    seg = half * _hmax(seg_v, lane16)[0]
    pe = jnp.where(pre[pl.ds(HALF - 16, 16)] == NEXT_TOKEN_ID,
                   jnp.int32(1), jnp.int32(0))
    prev_eq = half * pe[15]

    def scan_group(sh_eq, j, cnt, seg):
        rp = rowbase + jnp.int32(16) * j + lane16
        stt = sh_eq * jnp.where(rp != 0, jnp.int32(1), jnp.int32(0))
        ivec = _cumsum16(stt, lane16) + cnt
        iidx[pl.ds(16 * j, 16)] = ivec
        m = jnp.where(stt > 0, rp, jnp.int32(0))
        cm = jnp.maximum(_cummax16(m, lane16), seg)
        avec = rp - cm
        aidx[pl.ds(16 * j, 16)] = avec
        return ivec, avec, ivec[15], cm[15]

    meta = []
    for c in range(N_CHUNKS):
        j0 = (CHUNK // 16) * c
        if c == 0:
            sh0 = _rot(sv[pl.ds(0, 16)], lane16, 15)
            sh_eq0 = jnp.where(
                lane16 == 0, prev_eq,
                jnp.where(sh0 == NEXT_TOKEN_ID, jnp.int32(1), jnp.int32(0)))
        else:
            sh_eq0 = jnp.where(sv[pl.ds(16 * j0 - 1, 16)] == NEXT_TOKEN_ID,
                               jnp.int32(1), jnp.int32(0))
        ivec, avec, cnt, seg = scan_group(sh_eq0, jnp.int32(j0), cnt, seg)
        i0 = ivec[0]
        a0 = avec[0]

        def g_step(j, carry):
            cc, ss = carry
            sh_eq = jnp.where(
                sv[pl.ds(16 * j - 1, 16)] == NEXT_TOKEN_ID,
                jnp.int32(1), jnp.int32(0))
            _, _, cc, ss = scan_group(sh_eq, j, cc, ss)
            return cc, ss

        cnt, seg = lax.fori_loop(j0 + 1, j0 + CHUNK // 16, g_step,
                                 (cnt, seg))
        bnd = cnt - i0
        off = lax.rem(a0, jnp.int32(8))
        meta.append((i0, a0 - off, off, bnd))

    scat = {}

    def fire(c):
        s = c % NSLOT

        def grp(g, carry):
            tv = sv[pl.ds(c * CHUNK + 16 * g, 16)]
            for k in range(16):
                pltpu.async_copy(
                    tok_tab.at[pl.ds(tv[k], 1)],
                    tbufs[s].at[pl.ds(16 * g + k, 1)], tsems[s])
            return carry

        lax.fori_loop(0, CHUNK // 16, grp, jnp.int32(0))
        i0, argal, _off, bnd = meta[c]

        @pl.when(bnd == 0)
        def _fast():
            pltpu.async_copy(ins_tab.at[pl.ds(i0, 1)], inrs[s], asems[s])
            pltpu.async_copy(
                arg_tab.at[pl.ds(pl.multiple_of(argal, 8), CHUNK + 8)],
                abufs[s], asems[s])

        @pl.when(bnd != 0)
        def _slow():
            def grp2(g, carry):
                iv = iidx[pl.ds(c * CHUNK + 16 * g, 16)]
                av = aidx[pl.ds(c * CHUNK + 16 * g, 16)]
                for k in range(16):
                    r = 16 * g + k
                    pltpu.async_copy(
                        ins_tab.at[pl.ds(iv[k], 1)],
                        ibufs[s].at[pl.ds(r, 1)], asems[s])
                    pltpu.async_copy(
                        arg_tab.at[pl.ds(av[k], 1)],
                        abufs[s].at[pl.ds(r, 1)], asems[s])
                return carry

            lax.fori_loop(0, CHUNK // 16, grp2, jnp.int32(0))

    for c in range(min(NSLOT, N_CHUNKS)):
        fire(c)
    for c in range(N_CHUNKS):
        s = c % NSLOT
        pltpu.make_async_copy(
            tok_tab.at[pl.ds(0, CHUNK)], tbufs[s], tsems[s]).wait()
        _i0, _argal, off, bnd = meta[c]

        @pl.when(bnd == 0)
        def _drain_fast():
            pltpu.make_async_copy(
                ins_tab.at[pl.ds(0, 1)], inrs[s], asems[s]).wait()
            pltpu.make_async_copy(
                arg_tab.at[pl.ds(0, CHUNK + 8)], abufs[s], asems[s]).wait()

            iv8 = [inrs[s][0, pl.ds(16 * k, 16)] for k in range(NV)]

            def rowf(r, carry):
                xs = [tbufs[s][r, pl.ds(16 * k, 16)]
                      + abufs[s][r + off, pl.ds(16 * k, 16)] + iv8[k]
                      for k in range(NV)]
                _ln_store(tbufs[s], r, xs, wr, br, lane16)
                return carry

            lax.fori_loop(0, CHUNK, rowf, jnp.int32(0))

        @pl.when(bnd != 0)
        def _drain_slow():
            pltpu.make_async_copy(
                ins_tab.at[pl.ds(0, CHUNK)], ibufs[s], asems[s]).wait()
            pltpu.make_async_copy(
                arg_tab.at[pl.ds(0, CHUNK)],
                abufs[s].at[pl.ds(0, CHUNK)], asems[s]).wait()

            def rows(r, carry):
                xs = [tbufs[s][r, pl.ds(16 * k, 16)]
                      + abufs[s][r, pl.ds(16 * k, 16)]
                      + ibufs[s][r, pl.ds(16 * k, 16)]
                      for k in range(NV)]
                _ln_store(tbufs[s], r, xs, wr, br, lane16)
                return carry

            lax.fori_loop(0, CHUNK, rows, jnp.int32(0))

        scat[c] = pltpu.async_copy(
            tbufs[s], out_hbm.at[pl.ds(base + c * CHUNK, CHUNK)], osems[s])
        if c + NSLOT < N_CHUNKS:
            scat.pop(c).wait()
            fire(c + NSLOT)
    for c in sorted(scat):
        scat[c].wait()


_fused = functools.partial(
    pl.kernel,
    out_type=jax.ShapeDtypeStruct((N, D), jnp.float32),
    mesh=plsc.VectorSubcoreMesh(core_axis_name="c", subcore_axis_name="s"),
    scratch_types=[
        pltpu.VMEM((PER_W,), jnp.int32),
        pltpu.VMEM((HALF,), jnp.int32),
        pltpu.VMEM((PER_W,), jnp.int32),
        pltpu.VMEM((PER_W,), jnp.int32),
        pltpu.VMEM((D,), jnp.float32),
        pltpu.VMEM((D,), jnp.float32),
        [pltpu.VMEM((CHUNK, D), jnp.float32) for _ in range(NSLOT)],
        [pltpu.VMEM((CHUNK, D), jnp.float32) for _ in range(NSLOT)],
        [pltpu.VMEM((CHUNK + 8, D), jnp.float32) for _ in range(NSLOT)],
        [pltpu.VMEM((1, D), jnp.float32) for _ in range(NSLOT)],
        [pltpu.SemaphoreType.DMA for _ in range(NSLOT)],
        [pltpu.SemaphoreType.DMA for _ in range(NSLOT)],
        [pltpu.SemaphoreType.DMA for _ in range(NSLOT)],
    ],
)(_fused_kernel)


def kernel(state, token_table, instr_table, arg_table, ln_weight, ln_bias):
    arg_padded = jnp.concatenate(
        [arg_table, jnp.zeros((8, D), jnp.float32)], axis=0)
    out = _fused(state.reshape(N), token_table, instr_table, arg_padded,
                 ln_weight, ln_bias)
    return out.reshape(B, L, D)


# speedup vs baseline: 11.7200x; 1.0322x over previous
import functools

import jax
import jax.numpy as jnp
from jax import lax
from jax.experimental import pallas as pl
from jax.experimental.pallas import tpu as pltpu
from jax.experimental.pallas import tpu_sc as plsc

B = 16
L = 2048
D = 128
NEXT_TOKEN_ID = 5
EPS = 1e-05

N = B * L
NC = 2
NS = 16
NW = NC * NS
PER_W = N // NW
HALF = L // 2
CHUNK = 128
N_CHUNKS = PER_W // CHUNK
NSLOT = 2
NV = D // 16

_GDN = lax.GatherDimensionNumbers(
    offset_dims=(), collapsed_slice_dims=(0,), start_index_map=(0,))


def _rot(v, lane16, sh):
    perm = lax.rem(lane16 + jnp.int32(sh), jnp.int32(16))
    return lax.gather(v, perm[:, None], _GDN, (1,),
                      mode=lax.GatherScatterMode.PROMISE_IN_BOUNDS)


def _hsum(v, lane16):
    for sh in (8, 4, 2, 1):
        v = v + _rot(v, lane16, sh)
    return v


def _hmax(v, lane16):
    for sh in (8, 4, 2, 1):
        v = jnp.maximum(v, _rot(v, lane16, sh))
    return v


def _cumsum16(x, lane16):
    for sh in (1, 2, 4, 8):
        sl = _rot(x, lane16, 16 - sh)
        x = x + jnp.where(lane16 >= jnp.int32(sh), sl, jnp.int32(0))
    return x


def _cummax16(x, lane16):
    for sh in (1, 2, 4, 8):
        sl = _rot(x, lane16, 16 - sh)
        x = jnp.maximum(x, jnp.where(lane16 >= jnp.int32(sh), sl,
                                     jnp.int32(0)))
    return x


def _rsqrt_ladder(x):
    c = jnp.full(x.shape, 1.0, jnp.float32)
    for p in (8, 4, 2, 1):
        big = x >= jnp.float32(4.0 ** p)
        x = jnp.where(big, x * jnp.float32(4.0 ** -p), x)
        c = jnp.where(big, c * jnp.float32(2.0 ** -p), c)
    for p in (8, 4, 2, 1):
        small = x < jnp.float32(4.0 ** (1 - p))
        x = jnp.where(small, x * jnp.float32(4.0 ** p), x)
        c = jnp.where(small, c * jnp.float32(2.0 ** p), c)
    y = jnp.float32(1.27889018) + x * (
        jnp.float32(-0.3700147) + x * jnp.float32(0.04463214))
    for _ in range(3):
        y = y * (jnp.float32(1.5) - jnp.float32(0.5) * x * y * y)
    return y * c


def _ln_store(dst, r, xs, wr, br, lane16):
    tot = xs[0]
    sq = xs[0] * xs[0]
    for k in range(1, NV):
        tot = tot + xs[k]
        sq = sq + xs[k] * xs[k]
    inv_d = jnp.float32(1.0 / D)
    mean = _hsum(tot, lane16) * inv_d
    ex2 = _hsum(sq, lane16) * inv_d
    var = ex2 - mean * mean
    rstd = _rsqrt_ladder(var + jnp.float32(EPS))
    for k in range(NV):
        dst[r, pl.ds(16 * k, 16)] = (xs[k] - mean) * rstd * wr[k] + br[k]


def _fused_kernel(state_hbm, tok_tab, ins_tab, arg_tab, w_hbm, b_hbm,
                  out_hbm, sv, pre, iidx, aidx, wv, bv, tbufs, ibufs, abufs,
                  inrs, tsems, asems, osems):
    wid = lax.axis_index("s") * NC + lax.axis_index("c")
    base = pl.multiple_of(wid * PER_W, 256)
    half = lax.rem(wid, jnp.int32(2))
    rowbase = half * HALF
    lane16 = lax.broadcasted_iota(jnp.int32, (16,), 0)

    pltpu.sync_copy(state_hbm.at[pl.ds(base, PER_W)], sv)
    pre_base = pl.multiple_of(jnp.maximum(base - HALF, 0), 256)
    pltpu.sync_copy(state_hbm.at[pl.ds(pre_base, HALF)], pre)
    pltpu.sync_copy(w_hbm, wv)
    pltpu.sync_copy(b_hbm, bv)
    wr = [wv[pl.ds(16 * k, 16)] for k in range(NV)]
    br = [bv[pl.ds(16 * k, 16)] for k in range(NV)]

    def pre_step(j, carry):
        cnt_v, seg_v = carry
        rp = jnp.int32(16) * j + lane16
        e = (jnp.where(pre[pl.ds(16 * j, 16)] == NEXT_TOKEN_ID,
                       jnp.int32(1), jnp.int32(0))
             * jnp.where(rp <= jnp.int32(HALF - 2), jnp.int32(1),
                         jnp.int32(0)))
        cnt_v = cnt_v + e
        seg_v = jnp.maximum(seg_v, jnp.where(e > 0, rp + jnp.int32(1),
                                             jnp.int32(0)))
        return cnt_v, seg_v

    z = jnp.zeros((16,), jnp.int32)
    cnt_v, seg_v = lax.fori_loop(0, HALF // 16, pre_step, (z, z))
    cnt = half * _hsum(cnt_v, lane16)[0]
    seg = half * _hmax(seg_v, lane16)[0]
    pe = jnp.where(pre[pl.ds(HALF - 16, 16)] == NEXT_TOKEN_ID,
                   jnp.int32(1), jnp.int32(0))
    prev_eq = half * pe[15]

    def scan_group(sh_eq, j, cnt, seg):
        rp = rowbase + jnp.int32(16) * j + lane16
        stt = sh_eq * jnp.where(rp != 0, jnp.int32(1), jnp.int32(0))
        ivec = _cumsum16(stt, lane16) + cnt
        iidx[pl.ds(16 * j, 16)] = ivec
        m = jnp.where(stt > 0, rp, jnp.int32(0))
        cm = jnp.maximum(_cummax16(m, lane16), seg)
        avec = rp - cm
        aidx[pl.ds(16 * j, 16)] = avec
        return ivec, avec, ivec[15], cm[15]

    meta = []
    for c in range(N_CHUNKS):
        j0 = (CHUNK // 16) * c
        if c == 0:
            sh0 = _rot(sv[pl.ds(0, 16)], lane16, 15)
            sh_eq0 = jnp.where(
                lane16 == 0, prev_eq,
                jnp.where(sh0 == NEXT_TOKEN_ID, jnp.int32(1), jnp.int32(0)))
        else:
            sh_eq0 = jnp.where(sv[pl.ds(16 * j0 - 1, 16)] == NEXT_TOKEN_ID,
                               jnp.int32(1), jnp.int32(0))
        ivec, avec, cnt, seg = scan_group(sh_eq0, jnp.int32(j0), cnt, seg)
        i0 = ivec[0]
        a0 = avec[0]

        def g_step(j, carry):
            cc, ss = carry
            sh_eq = jnp.where(
                sv[pl.ds(16 * j - 1, 16)] == NEXT_TOKEN_ID,
                jnp.int32(1), jnp.int32(0))
            _, _, cc, ss = scan_group(sh_eq, j, cc, ss)
            return cc, ss

        cnt, seg = lax.fori_loop(j0 + 1, j0 + CHUNK // 16, g_step,
                                 (cnt, seg))
        bnd = cnt - i0
        off = lax.rem(a0, jnp.int32(8))
        meta.append((i0, a0 - off, off, bnd))

    scat = {}

    def fire(c):
        s = c % NSLOT

        def grp(g, carry):
            tv = sv[pl.ds(c * CHUNK + 16 * g, 16)]
            for k in range(16):
                pltpu.async_copy(
                    tok_tab.at[pl.ds(tv[k], 1)],
                    tbufs[s].at[pl.ds(16 * g + k, 1)], tsems[s])
            return carry

        lax.fori_loop(0, CHUNK // 16, grp, jnp.int32(0))
        if c - NSLOT in scat:
            scat.pop(c - NSLOT).wait()
        i0, argal, _off, bnd = meta[c]

        @pl.when(bnd == 0)
        def _fast():
            pltpu.async_copy(ins_tab.at[pl.ds(i0, 1)], inrs[s], asems[s])
            pltpu.async_copy(
                arg_tab.at[pl.ds(pl.multiple_of(argal, 8), CHUNK + 8)],
                abufs[s], asems[s])

        @pl.when(bnd != 0)
        def _slow():
            def grp2(g, carry):
                iv = iidx[pl.ds(c * CHUNK + 16 * g, 16)]
                av = aidx[pl.ds(c * CHUNK + 16 * g, 16)]
                for k in range(16):
                    r = 16 * g + k
                    pltpu.async_copy(
                        ins_tab.at[pl.ds(iv[k], 1)],
                        ibufs[s].at[pl.ds(r, 1)], asems[s])
                    pltpu.async_copy(
                        arg_tab.at[pl.ds(av[k], 1)],
                        abufs[s].at[pl.ds(r, 1)], asems[s])
                return carry

            lax.fori_loop(0, CHUNK // 16, grp2, jnp.int32(0))

    for c in range(min(NSLOT, N_CHUNKS)):
        fire(c)
    for c in range(N_CHUNKS):
        s = c % NSLOT
        pltpu.make_async_copy(
            tok_tab.at[pl.ds(0, CHUNK)], tbufs[s], tsems[s]).wait()
        _i0, _argal, off, bnd = meta[c]

        @pl.when(bnd == 0)
        def _drain_fast():
            pltpu.make_async_copy(
                ins_tab.at[pl.ds(0, 1)], inrs[s], asems[s]).wait()
            pltpu.make_async_copy(
                arg_tab.at[pl.ds(0, CHUNK + 8)], abufs[s], asems[s]).wait()

            iv8 = [inrs[s][0, pl.ds(16 * k, 16)] for k in range(NV)]

            def rowf(r, carry):
                for u in range(2):
                    ru = jnp.int32(2) * r + jnp.int32(u)
                    xs = [tbufs[s][ru, pl.ds(16 * k, 16)]
                          + abufs[s][ru + off, pl.ds(16 * k, 16)] + iv8[k]
                          for k in range(NV)]
                    _ln_store(abufs[s], ru, xs, wr, br, lane16)
                return carry

            lax.fori_loop(0, CHUNK // 2, rowf, jnp.int32(0))

        @pl.when(bnd != 0)
        def _drain_slow():
            pltpu.make_async_copy(
                ins_tab.at[pl.ds(0, CHUNK)], ibufs[s], asems[s]).wait()
            pltpu.make_async_copy(
                arg_tab.at[pl.ds(0, CHUNK)],
                abufs[s].at[pl.ds(0, CHUNK)], asems[s]).wait()

            def rows(r, carry):
                for u in range(2):
                    ru = jnp.int32(2) * r + jnp.int32(u)
                    xs = [tbufs[s][ru, pl.ds(16 * k, 16)]
                          + abufs[s][ru, pl.ds(16 * k, 16)]
                          + ibufs[s][ru, pl.ds(16 * k, 16)]
                          for k in range(NV)]
                    _ln_store(abufs[s], ru, xs, wr, br, lane16)
                return carry

            lax.fori_loop(0, CHUNK // 2, rows, jnp.int32(0))

        scat[c] = pltpu.async_copy(
            abufs[s].at[pl.ds(0, CHUNK)],
            out_hbm.at[pl.ds(base + c * CHUNK, CHUNK)], osems[s])
        if c + NSLOT < N_CHUNKS:
            fire(c + NSLOT)
    for c in sorted(scat):
        scat[c].wait()


_fused = functools.partial(
    pl.kernel,
    out_type=jax.ShapeDtypeStruct((N, D), jnp.float32),
    mesh=plsc.VectorSubcoreMesh(core_axis_name="c", subcore_axis_name="s"),
    scratch_types=[
        pltpu.VMEM((PER_W,), jnp.int32),
        pltpu.VMEM((HALF,), jnp.int32),
        pltpu.VMEM((PER_W,), jnp.int32),
        pltpu.VMEM((PER_W,), jnp.int32),
        pltpu.VMEM((D,), jnp.float32),
        pltpu.VMEM((D,), jnp.float32),
        [pltpu.VMEM((CHUNK, D), jnp.float32) for _ in range(NSLOT)],
        [pltpu.VMEM((CHUNK, D), jnp.float32) for _ in range(NSLOT)],
        [pltpu.VMEM((CHUNK + 8, D), jnp.float32) for _ in range(NSLOT)],
        [pltpu.VMEM((1, D), jnp.float32) for _ in range(NSLOT)],
        [pltpu.SemaphoreType.DMA for _ in range(NSLOT)],
        [pltpu.SemaphoreType.DMA for _ in range(NSLOT)],
        [pltpu.SemaphoreType.DMA for _ in range(NSLOT)],
    ],
)(_fused_kernel)


def kernel(state, token_table, instr_table, arg_table, ln_weight, ln_bias):
    arg_padded = jnp.concatenate(
        [arg_table, jnp.zeros((8, D), jnp.float32)], axis=0)
    out = _fused(state.reshape(N), token_table, instr_table, arg_padded,
                 ln_weight, ln_bias)
    return out.reshape(B, L, D)


# speedup vs baseline: 11.7497x; 1.0025x over previous
import functools

import jax
import jax.numpy as jnp
from jax import lax
from jax.experimental import pallas as pl
from jax.experimental.pallas import tpu as pltpu
from jax.experimental.pallas import tpu_sc as plsc

B = 16
L = 2048
D = 128
NEXT_TOKEN_ID = 5
EPS = 1e-05

N = B * L
NC = 2
NS = 16
NW = NC * NS
PER_W = N // NW
HALF = L // 2
CHUNK = 128
N_CHUNKS = PER_W // CHUNK
NSLOT = 2
NV = D // 16

_GDN = lax.GatherDimensionNumbers(
    offset_dims=(), collapsed_slice_dims=(0,), start_index_map=(0,))


def _rot(v, lane16, sh):
    perm = lax.rem(lane16 + jnp.int32(sh), jnp.int32(16))
    return lax.gather(v, perm[:, None], _GDN, (1,),
                      mode=lax.GatherScatterMode.PROMISE_IN_BOUNDS)


def _hsum(v, lane16):
    for sh in (8, 4, 2, 1):
        v = v + _rot(v, lane16, sh)
    return v


def _hmax(v, lane16):
    for sh in (8, 4, 2, 1):
        v = jnp.maximum(v, _rot(v, lane16, sh))
    return v


def _cumsum16(x, lane16):
    for sh in (1, 2, 4, 8):
        sl = _rot(x, lane16, 16 - sh)
        x = x + jnp.where(lane16 >= jnp.int32(sh), sl, jnp.int32(0))
    return x


def _cummax16(x, lane16):
    for sh in (1, 2, 4, 8):
        sl = _rot(x, lane16, 16 - sh)
        x = jnp.maximum(x, jnp.where(lane16 >= jnp.int32(sh), sl,
                                     jnp.int32(0)))
    return x


def _rsqrt_ladder(x):
    c = jnp.full(x.shape, 1.0, jnp.float32)
    for p in (8, 4, 2, 1):
        big = x >= jnp.float32(4.0 ** p)
        x = jnp.where(big, x * jnp.float32(4.0 ** -p), x)
        c = jnp.where(big, c * jnp.float32(2.0 ** -p), c)
    for p in (8, 4, 2, 1):
        small = x < jnp.float32(4.0 ** (1 - p))
        x = jnp.where(small, x * jnp.float32(4.0 ** p), x)
        c = jnp.where(small, c * jnp.float32(2.0 ** p), c)
    y = jnp.float32(1.27889018) + x * (
        jnp.float32(-0.3700147) + x * jnp.float32(0.04463214))
    for _ in range(2):
        y = y * (jnp.float32(1.5) - jnp.float32(0.5) * x * y * y)
    return y * c


def _ln_store(dst, r, xs, wr, br, lane16):
    tot = xs[0]
    sq = xs[0] * xs[0]
    for k in range(1, NV):
        tot = tot + xs[k]
        sq = sq + xs[k] * xs[k]
    inv_d = jnp.float32(1.0 / D)
    mean = _hsum(tot, lane16) * inv_d
    ex2 = _hsum(sq, lane16) * inv_d
    var = ex2 - mean * mean
    rstd = _rsqrt_ladder(var + jnp.float32(EPS))
    for k in range(NV):
        dst[r, pl.ds(16 * k, 16)] = (xs[k] - mean) * rstd * wr[k] + br[k]


def _fused_kernel(state_hbm, tok_tab, ins_tab, arg_tab, w_hbm, b_hbm,
                  out_hbm, sv, pre, iidx, aidx, wv, bv, tbufs, ibufs, abufs,
                  inrs, tsems, asems, osems):
    wid = lax.axis_index("s") * NC + lax.axis_index("c")
    base = pl.multiple_of(wid * PER_W, 256)
    half = lax.rem(wid, jnp.int32(2))
    rowbase = half * HALF
    lane16 = lax.broadcasted_iota(jnp.int32, (16,), 0)

    pltpu.sync_copy(state_hbm.at[pl.ds(base, PER_W)], sv)
    pre_base = pl.multiple_of(jnp.maximum(base - HALF, 0), 256)
    pltpu.sync_copy(state_hbm.at[pl.ds(pre_base, HALF)], pre)
    pltpu.sync_copy(w_hbm, wv)
    pltpu.sync_copy(b_hbm, bv)
    wr = [wv[pl.ds(16 * k, 16)] for k in range(NV)]
    br = [bv[pl.ds(16 * k, 16)] for k in range(NV)]

    def pre_step(j, carry):
        cnt_v, seg_v = carry
        rp = jnp.int32(16) * j + lane16
        e = (jnp.where(pre[pl.ds(16 * j, 16)] == NEXT_TOKEN_ID,
                       jnp.int32(1), jnp.int32(0))
             * jnp.where(rp <= jnp.int32(HALF - 2), jnp.int32(1),
                         jnp.int32(0)))
        cnt_v = cnt_v + e
        seg_v = jnp.maximum(seg_v, jnp.where(e > 0, rp + jnp.int32(1),
                                             jnp.int32(0)))
        return cnt_v, seg_v

    z = jnp.zeros((16,), jnp.int32)
    cnt_v, seg_v = lax.fori_loop(0, HALF // 16, pre_step, (z, z))
    cnt = half * _hsum(cnt_v, lane16)[0]
    seg = half * _hmax(seg_v, lane16)[0]
    pe = jnp.where(pre[pl.ds(HALF - 16, 16)] == NEXT_TOKEN_ID,
                   jnp.int32(1), jnp.int32(0))
    prev_eq = half * pe[15]

    def scan_group(sh_eq, j, cnt, seg):
        rp = rowbase + jnp.int32(16) * j + lane16
        stt = sh_eq * jnp.where(rp != 0, jnp.int32(1), jnp.int32(0))
        ivec = _cumsum16(stt, lane16) + cnt
        iidx[pl.ds(16 * j, 16)] = ivec
        m = jnp.where(stt > 0, rp, jnp.int32(0))
        cm = jnp.maximum(_cummax16(m, lane16), seg)
        avec = rp - cm
        aidx[pl.ds(16 * j, 16)] = avec
        return ivec, avec, ivec[15], cm[15]

    meta = []
    for c in range(N_CHUNKS):
        j0 = (CHUNK // 16) * c
        if c == 0:
            sh0 = _rot(sv[pl.ds(0, 16)], lane16, 15)
            sh_eq0 = jnp.where(
                lane16 == 0, prev_eq,
                jnp.where(sh0 == NEXT_TOKEN_ID, jnp.int32(1), jnp.int32(0)))
        else:
            sh_eq0 = jnp.where(sv[pl.ds(16 * j0 - 1, 16)] == NEXT_TOKEN_ID,
                               jnp.int32(1), jnp.int32(0))
        ivec, avec, cnt, seg = scan_group(sh_eq0, jnp.int32(j0), cnt, seg)
        i0 = ivec[0]
        a0 = avec[0]

        def g_step(j, carry):
            cc, ss = carry
            sh_eq = jnp.where(
                sv[pl.ds(16 * j - 1, 16)] == NEXT_TOKEN_ID,
                jnp.int32(1), jnp.int32(0))
            _, _, cc, ss = scan_group(sh_eq, j, cc, ss)
            return cc, ss

        cnt, seg = lax.fori_loop(j0 + 1, j0 + CHUNK // 16, g_step,
                                 (cnt, seg))
        bnd = cnt - i0
        off = lax.rem(a0, jnp.int32(8))
        meta.append((i0, a0 - off, off, bnd))

    scat = {}

    def fire(c):
        s = c % NSLOT

        def grp(g, carry):
            tv = sv[pl.ds(c * CHUNK + 16 * g, 16)]
            for k in range(16):
                pltpu.async_copy(
                    tok_tab.at[pl.ds(tv[k], 1)],
                    tbufs[s].at[pl.ds(16 * g + k, 1)], tsems[s])
            return carry

        lax.fori_loop(0, CHUNK // 16, grp, jnp.int32(0))
        if c - NSLOT in scat:
            scat.pop(c - NSLOT).wait()
        i0, argal, _off, bnd = meta[c]

        @pl.when(bnd == 0)
        def _fast():
            pltpu.async_copy(ins_tab.at[pl.ds(i0, 1)], inrs[s], asems[s])
            pltpu.async_copy(
                arg_tab.at[pl.ds(pl.multiple_of(argal, 8), CHUNK + 8)],
                abufs[s], asems[s])

        @pl.when(bnd != 0)
        def _slow():
            def grp2(g, carry):
                iv = iidx[pl.ds(c * CHUNK + 16 * g, 16)]
                av = aidx[pl.ds(c * CHUNK + 16 * g, 16)]
                for k in range(16):
                    r = 16 * g + k
                    pltpu.async_copy(
                        ins_tab.at[pl.ds(iv[k], 1)],
                        ibufs[s].at[pl.ds(r, 1)], asems[s])
                    pltpu.async_copy(
                        arg_tab.at[pl.ds(av[k], 1)],
                        abufs[s].at[pl.ds(r, 1)], asems[s])
                return carry

            lax.fori_loop(0, CHUNK // 16, grp2, jnp.int32(0))

    for c in range(min(NSLOT, N_CHUNKS)):
        fire(c)
    for c in range(N_CHUNKS):
        s = c % NSLOT
        pltpu.make_async_copy(
            tok_tab.at[pl.ds(0, CHUNK)], tbufs[s], tsems[s]).wait()
        _i0, _argal, off, bnd = meta[c]

        @pl.when(bnd == 0)
        def _drain_fast():
            pltpu.make_async_copy(
                ins_tab.at[pl.ds(0, 1)], inrs[s], asems[s]).wait()
            pltpu.make_async_copy(
                arg_tab.at[pl.ds(0, CHUNK + 8)], abufs[s], asems[s]).wait()

            iv8 = [inrs[s][0, pl.ds(16 * k, 16)] for k in range(NV)]

            def rowf(r, carry):
                for u in range(2):
                    ru = jnp.int32(2) * r + jnp.int32(u)
                    xs = [tbufs[s][ru, pl.ds(16 * k, 16)]
                          + abufs[s][ru + off, pl.ds(16 * k, 16)] + iv8[k]
                          for k in range(NV)]
                    _ln_store(abufs[s], ru, xs, wr, br, lane16)
                return carry

            lax.fori_loop(0, CHUNK // 2, rowf, jnp.int32(0))

        @pl.when(bnd != 0)
        def _drain_slow():
            pltpu.make_async_copy(
                ins_tab.at[pl.ds(0, CHUNK)], ibufs[s], asems[s]).wait()
            pltpu.make_async_copy(
                arg_tab.at[pl.ds(0, CHUNK)],
                abufs[s].at[pl.ds(0, CHUNK)], asems[s]).wait()

            def rows(r, carry):
                for u in range(2):
                    ru = jnp.int32(2) * r + jnp.int32(u)
                    xs = [tbufs[s][ru, pl.ds(16 * k, 16)]
                          + abufs[s][ru, pl.ds(16 * k, 16)]
                          + ibufs[s][ru, pl.ds(16 * k, 16)]
                          for k in range(NV)]
                    _ln_store(abufs[s], ru, xs, wr, br, lane16)
                return carry

            lax.fori_loop(0, CHUNK // 2, rows, jnp.int32(0))

        scat[c] = pltpu.async_copy(
            abufs[s].at[pl.ds(0, CHUNK)],
            out_hbm.at[pl.ds(base + c * CHUNK, CHUNK)], osems[s])
        if c + NSLOT < N_CHUNKS:
            fire(c + NSLOT)
    for c in sorted(scat):
        scat[c].wait()


_fused = functools.partial(
    pl.kernel,
    out_type=jax.ShapeDtypeStruct((N, D), jnp.float32),
    mesh=plsc.VectorSubcoreMesh(core_axis_name="c", subcore_axis_name="s"),
    scratch_types=[
        pltpu.VMEM((PER_W,), jnp.int32),
        pltpu.VMEM((HALF,), jnp.int32),
        pltpu.VMEM((PER_W,), jnp.int32),
        pltpu.VMEM((PER_W,), jnp.int32),
        pltpu.VMEM((D,), jnp.float32),
        pltpu.VMEM((D,), jnp.float32),
        [pltpu.VMEM((CHUNK, D), jnp.float32) for _ in range(NSLOT)],
        [pltpu.VMEM((CHUNK, D), jnp.float32) for _ in range(NSLOT)],
        [pltpu.VMEM((CHUNK + 8, D), jnp.float32) for _ in range(NSLOT)],
        [pltpu.VMEM((1, D), jnp.float32) for _ in range(NSLOT)],
        [pltpu.SemaphoreType.DMA for _ in range(NSLOT)],
        [pltpu.SemaphoreType.DMA for _ in range(NSLOT)],
        [pltpu.SemaphoreType.DMA for _ in range(NSLOT)],
    ],
)(_fused_kernel)


def kernel(state, token_table, instr_table, arg_table, ln_weight, ln_bias):
    arg_padded = jnp.concatenate(
        [arg_table, jnp.zeros((8, D), jnp.float32)], axis=0)
    out = _fused(state.reshape(N), token_table, instr_table, arg_padded,
                 ln_weight, ln_bias)
    return out.reshape(B, L, D)
